# Initial kernel scaffold; baseline (speedup 1.0000x reference)
import jax
import jax.numpy as jnp
from jax.experimental import pallas as pl


def kernel(query, reference_points, input_flatten, input_spatial_shapes, input_level_start_index, W_value, b_value, W_key, b_key, W_query, b_query, W_off, b_off, W_out, b_out):
    raise NotImplementedError("write your pallas kernel here")


# speedup vs baseline: 37.1409x; 37.1409x over previous
import functools
import math

import jax
import jax.numpy as jnp
from jax import lax
from jax.experimental import pallas as pl
from jax.experimental.pallas import tpu as pltpu
from jax.experimental.pallas import tpu_sc as plsc

_NH, _NL, _NP, _DH = 8, 4, 4, 32
_NW = 32
_CH = 1024
_SUB = 128


def _proj_vk_body(x_ref, wv_ref, bv_ref, wk_ref, bk_ref, v_ref, k_ref):
    x = x_ref[0]
    v_ref[0] = jnp.dot(x, wv_ref[...], preferred_element_type=jnp.float32) + bv_ref[...]
    k_ref[0] = jnp.dot(x, wk_ref[...], preferred_element_type=jnp.float32) + bk_ref[...]


def _query_body(lin, q_ref, wq_ref, bq_ref, wo_ref, bo_ref, ref_ref, cst_ref,
                qp_ref, idx_ref, w_ref):
    b = pl.program_id(0)
    q = q_ref[0]
    qp_ref[0] = jnp.dot(q, wq_ref[...], preferred_element_type=jnp.float32) + bq_ref[...]
    off = jnp.dot(q, wo_ref[...], preferred_element_type=jnp.float32) + bo_ref[...]
    sz = cst_ref[0:1, :]
    g = ref_ref[0] * sz + off - 0.5
    gx = g[:, :128]
    gy = g[:, 128:]
    wc = cst_ref[0:1, 0:128]
    hc = cst_ref[0:1, 128:256]
    st = cst_ref[1:2, 0:128]
    mc = cst_ref[2:3, 0:128]
    x0 = jnp.floor(gx)
    y0 = jnp.floor(gy)
    fx = gx - x0
    fy = gy - y0
    x1 = x0 + 1.0
    y1 = y0 + 1.0
    basef = b.astype(jnp.float32) * float(lin) + st

    def ridx(xf, yf):
        xc = jnp.clip(xf, 0.0, wc - 1.0)
        yc = jnp.clip(yf, 0.0, hc - 1.0)
        return ((basef + yc * wc + xc) * 8.0 + mc).astype(jnp.int32)

    def vmask(xf, yf):
        ok = (xf >= 0.0) & (xf <= wc - 1.0) & (yf >= 0.0) & (yf <= hc - 1.0)
        return ok.astype(jnp.float32)

    idx_ref[0, 0] = ridx(x0, y0)
    w_ref[0, 0] = (1.0 - fx) * (1.0 - fy) * vmask(x0, y0)
    idx_ref[0, 1] = ridx(x1, y0)
    w_ref[0, 1] = fx * (1.0 - fy) * vmask(x1, y0)
    idx_ref[0, 2] = ridx(x0, y1)
    w_ref[0, 2] = (1.0 - fx) * fy * vmask(x0, y1)
    idx_ref[0, 3] = ridx(x1, y1)
    w_ref[0, 3] = fx * fy * vmask(x1, y1)


def _sc_gather_body(r4, idx_hbm, vt_hbm, kt_hbm, outv, outk,
                    idx_v, bufv, bufk, semv, semk):
    wid = lax.axis_index("s") * 2 + lax.axis_index("c")
    rows_w = r4 // _NW
    n_ch = rows_w // _CH
    base = wid * rows_w

    def body(i, carry):
        off = pl.multiple_of(base + i * _CH, _CH)
        roff = pl.multiple_of(off // 128, _CH // 128)
        pltpu.sync_copy(idx_hbm.at[pl.ds(roff, _CH // 128)], idx_v)
        cps = []
        for j in range(_CH // _SUB):
            sl = pl.ds(j * _SUB, _SUB)
            cps.append(pltpu.async_copy(vt_hbm.at[idx_v.at[j]], bufv.at[sl], semv))
            cps.append(pltpu.async_copy(kt_hbm.at[idx_v.at[j]], bufk.at[sl], semk))
        for c in cps:
            c.wait()
        pltpu.sync_copy(bufv, outv.at[pl.ds(off, _CH)])
        pltpu.sync_copy(bufk, outk.at[pl.ds(off, _CH)])
        return carry

    lax.fori_loop(0, n_ch, body, 0)


def _attn_body(k4_ref, v4_ref, w_ref, q_ref, wo_ref, bo_ref, o_ref):
    blk = q_ref.shape[1]
    f32 = jnp.float32
    isq = 1.0 / math.sqrt(float(_DH))
    i32 = jnp.int32
    e_sel = (lax.broadcasted_iota(i32, (512, 16), 0) // _DH
             == lax.broadcasted_iota(i32, (512, 16), 1)).astype(f32)
    e_selt = (lax.broadcasted_iota(i32, (16, 512), 1) // _DH
              == lax.broadcasted_iota(i32, (16, 512), 0)).astype(f32)
    f_sel = (lax.broadcasted_iota(i32, (512, _DH), 0) % _DH
             == lax.broadcasted_iota(i32, (512, _DH), 1)).astype(f32)
    k4 = k4_ref[0]
    v4 = v4_ref[0]
    w = w_ref[0]
    q = q_ref[0]
    outs = []
    for m in range(_NH):
        qt = jnp.tile(q[:, m * _DH:(m + 1) * _DH], (1, 16))
        prod = (k4[:, :, m] * qt[None, :, :]).reshape(4 * blk, 512)
        dots = jnp.dot(prod, e_sel, preferred_element_type=f32) * isq
        wm = w[:, :, m]
        lg = (wm * dots.reshape(4, blk, 16)).sum(axis=0)
        mx = jnp.max(lg, axis=-1, keepdims=True)
        ex = jnp.exp(lg - mx)
        at = ex / jnp.sum(ex, axis=-1, keepdims=True)
        aw = (wm * at[None, :, :]).reshape(4 * blk, 16)
        aw512 = jnp.dot(aw, e_selt, preferred_element_type=f32)
        sv = (aw512 * v4[:, :, m].reshape(4 * blk, 512)).reshape(4, blk, 512).sum(axis=0)
        outs.append(jnp.dot(sv, f_sel, preferred_element_type=f32))
    row = jnp.concatenate(outs, axis=-1)
    o_ref[0] = jnp.dot(row, wo_ref[...], preferred_element_type=f32) + bo_ref[...]


def _sc_gather(idx2, vt, kt, r4):
    mesh = plsc.VectorSubcoreMesh(core_axis_name="c", subcore_axis_name="s",
                                  num_cores=2, num_subcores=16)
    f32 = jnp.float32
    run = pl.kernel(
        functools.partial(_sc_gather_body, r4),
        out_type=(jax.ShapeDtypeStruct((r4, _DH), f32),
                  jax.ShapeDtypeStruct((r4, _DH), f32)),
        mesh=mesh,
        scratch_types=[
            pltpu.VMEM((_CH // 128, 128), jnp.int32),
            pltpu.VMEM((_CH, _DH), f32),
            pltpu.VMEM((_CH, _DH), f32),
            pltpu.SemaphoreType.DMA,
            pltpu.SemaphoreType.DMA,
        ],
        compiler_params=pltpu.CompilerParams(use_tc_tiling_on_sc=False),
    )
    return run(idx2, vt, kt)


def kernel(query, reference_points, input_flatten, input_spatial_shapes,
           input_level_start_index, W_value, b_value, W_key, b_key,
           W_query, b_query, W_off, b_off, W_out, b_out):
    f32 = jnp.float32
    B, LQ, DM = query.shape
    LIN = input_flatten.shape[1]
    M, L, P, D = _NH, _NL, _NP, _DH

    Wo_p = W_off.reshape(DM, M, L, P, 2).transpose(0, 4, 1, 2, 3).reshape(DM, DM)
    bo_p = b_off.reshape(M, L, P, 2).transpose(3, 0, 1, 2).reshape(1, DM)
    shp = input_spatial_shapes.astype(f32)
    Wl, Hl = shp[:, 1], shp[:, 0]
    stl = input_level_start_index.astype(f32)
    W128 = jnp.tile(jnp.repeat(Wl, P), (M,))
    H128 = jnp.tile(jnp.repeat(Hl, P), (M,))
    S128 = jnp.tile(jnp.repeat(stl, P), (M,))
    M128 = jnp.repeat(jnp.arange(M, dtype=f32), L * P)
    cst = jnp.zeros((8, 2 * 128), f32)
    cst = cst.at[0, :128].set(W128).at[0, 128:].set(H128)
    cst = cst.at[1, :128].set(S128).at[1, 128:].set(S128)
    cst = cst.at[2, :128].set(M128).at[2, 128:].set(M128)
    refx = jnp.tile(jnp.repeat(reference_points[..., 0], P, axis=-1), (1, 1, M))
    refy = jnp.tile(jnp.repeat(reference_points[..., 1], P, axis=-1), (1, 1, M))
    refe = jnp.concatenate([refx, refy], axis=-1)
    bv2 = b_value.reshape(1, DM)
    bk2 = b_key.reshape(1, DM)
    bq2 = b_query.reshape(1, DM)
    bo2 = b_out.reshape(1, DM)

    BLK_I = 544
    vproj, kproj = pl.pallas_call(
        _proj_vk_body,
        grid=(B, LIN // BLK_I),
        in_specs=[
            pl.BlockSpec((1, BLK_I, DM), lambda b, i: (b, i, 0)),
            pl.BlockSpec((DM, DM), lambda b, i: (0, 0)),
            pl.BlockSpec((1, DM), lambda b, i: (0, 0)),
            pl.BlockSpec((DM, DM), lambda b, i: (0, 0)),
            pl.BlockSpec((1, DM), lambda b, i: (0, 0)),
        ],
        out_specs=[pl.BlockSpec((1, BLK_I, DM), lambda b, i: (b, i, 0))] * 2,
        out_shape=[jax.ShapeDtypeStruct((B, LIN, DM), f32)] * 2,
    )(input_flatten, W_value, bv2, W_key, bk2)

    BLK_Q = 512
    qp, idx4, w4 = pl.pallas_call(
        functools.partial(_query_body, LIN),
        grid=(B, LQ // BLK_Q),
        in_specs=[
            pl.BlockSpec((1, BLK_Q, DM), lambda b, i: (b, i, 0)),
            pl.BlockSpec((DM, DM), lambda b, i: (0, 0)),
            pl.BlockSpec((1, DM), lambda b, i: (0, 0)),
            pl.BlockSpec((DM, DM), lambda b, i: (0, 0)),
            pl.BlockSpec((1, DM), lambda b, i: (0, 0)),
            pl.BlockSpec((1, BLK_Q, DM), lambda b, i: (b, i, 0)),
            pl.BlockSpec((8, 2 * 128), lambda b, i: (0, 0)),
        ],
        out_specs=[
            pl.BlockSpec((1, BLK_Q, DM), lambda b, i: (b, i, 0)),
            pl.BlockSpec((1, 4, BLK_Q, 128), lambda b, i: (b, 0, i, 0)),
            pl.BlockSpec((1, 4, BLK_Q, 128), lambda b, i: (b, 0, i, 0)),
        ],
        out_shape=[
            jax.ShapeDtypeStruct((B, LQ, DM), f32),
            jax.ShapeDtypeStruct((B, 4, LQ, 128), jnp.int32),
            jax.ShapeDtypeStruct((B, 4, LQ, 128), f32),
        ],
    )(query, W_query, bq2, Wo_p, bo_p, refe, cst)

    R4 = B * LQ * 4 * M * L * P
    idx_t = idx4.reshape(R4 // 128, 128)
    vt = vproj.reshape(B * LIN * M, D)
    kt = kproj.reshape(B * LIN * M, D)

    V4, K4 = _sc_gather(idx_t, vt, kt, R4)
    V4c = V4.reshape(B, 4, LQ, M, 16 * D)
    K4c = K4.reshape(B, 4, LQ, M, 16 * D)
    w_t = w4.reshape(B, 4, LQ, M, 16)

    BLK_C = 128
    out = pl.pallas_call(
        _attn_body,
        grid=(B, LQ // BLK_C),
        in_specs=[
            pl.BlockSpec((1, 4, BLK_C, M, 16 * D), lambda b, i: (b, 0, i, 0, 0)),
            pl.BlockSpec((1, 4, BLK_C, M, 16 * D), lambda b, i: (b, 0, i, 0, 0)),
            pl.BlockSpec((1, 4, BLK_C, M, 16), lambda b, i: (b, 0, i, 0, 0)),
            pl.BlockSpec((1, BLK_C, DM), lambda b, i: (b, i, 0)),
            pl.BlockSpec((DM, DM), lambda b, i: (0, 0)),
            pl.BlockSpec((1, DM), lambda b, i: (0, 0)),
        ],
        out_specs=pl.BlockSpec((1, BLK_C, DM), lambda b, i: (b, i, 0)),
        out_shape=jax.ShapeDtypeStruct((B, LQ, DM), f32),
    )(K4c, V4c, w_t, qp, W_out, bo2)
    return out
